# baseline (speedup 1.0000x reference)
import functools

import jax
import jax.numpy as jnp
from jax import lax
from jax.experimental import pallas as pl
from jax.experimental.pallas import tpu as pltpu
from jax.experimental.pallas import tpu_sc as plsc

B = 16384
C = 5
E = 64
NC = 2
NS = 16
NW = NC * NS
B_PER_W = B // NW
CB = 128
N_CHUNK = B_PER_W // CB


def _sc_kernel(th_hbm, toff_hbm, ch_hbm, coff_hbm, ttab_hbm, ctab_hbm,
               out_hbm, th_v, toff_v, ch_v, coff_v, w_rows, c_rows, out_v,
               sem):
    wid = lax.axis_index("s") * NC + lax.axis_index("c")
    base = wid * B_PER_W

    def chunk_body(ch, _):
        tbase = base + ch * CB

        pltpu.sync_copy(th_hbm.at[pl.ds(tbase, CB)], th_v)
        pltpu.sync_copy(toff_hbm.at[pl.ds(tbase, CB)], toff_v)
        pltpu.sync_copy(ch_hbm.at[pl.ds(tbase * C, CB * C)], ch_v)
        pltpu.sync_copy(coff_hbm.at[pl.ds(tbase * C, CB * C)], coff_v)

        cps = [pltpu.async_copy(ttab_hbm.at[th_v], w_rows, sem)]
        for j in range(C):
            cps.append(pltpu.async_copy(
                ctab_hbm.at[ch_v.at[pl.ds(j * CB, CB)]],
                c_rows.at[pl.ds(j * CB, CB)], sem))
        for cp in cps:
            cp.wait()

        iota16 = lax.iota(jnp.int32, 16)

        def group_body(g, _):
            b_vec = iota16 + g * 16
            b5 = b_vec * C
            woff = toff_v[pl.ds(g * 16, 16)]
            coffs = [plsc.load_gather(coff_v, [b5 + c]) for c in range(C)]

            def e_body(e, accs):
                evec = jnp.zeros((16,), jnp.int32) + e
                wv = plsc.load_gather(w_rows, [b_vec, woff + evec])
                return tuple(
                    accs[c] + wv * plsc.load_gather(
                        c_rows, [b5 + c, coffs[c] + evec])
                    for c in range(C))

            accs = lax.fori_loop(
                0, E, e_body,
                tuple(jnp.zeros((16,), jnp.float32) for _ in range(C)),
                unroll=4)
            for c in range(C):
                plsc.store_scatter(out_v, [b5 + c], accs[c])
            return 0

        lax.fori_loop(0, CB // 16, group_body, 0)
        pltpu.sync_copy(out_v, out_hbm.at[pl.ds(tbase * C, CB * C)])
        return 0

    lax.fori_loop(0, N_CHUNK, chunk_body, 0)


@jax.jit
def _run(target, ctx_flat, target_table, context_table):
    t_half = lax.shift_right_logical(target, 1)
    t_off = (target & 1) * E
    c_half = lax.shift_right_logical(ctx_flat, 1)
    c_off = (ctx_flat & 1) * E
    ttab2 = target_table.reshape(-1, 2 * E)
    ctab2 = context_table.reshape(-1, 2 * E)

    mesh = plsc.VectorSubcoreMesh(core_axis_name="c", subcore_axis_name="s")
    k = functools.partial(
        pl.kernel, mesh=mesh,
        out_type=jax.ShapeDtypeStruct((B * C,), jnp.float32),
        compiler_params=pltpu.CompilerParams(needs_layout_passes=False),
        scratch_types=[
            pltpu.VMEM((CB,), jnp.int32),
            pltpu.VMEM((CB,), jnp.int32),
            pltpu.VMEM((CB * C,), jnp.int32),
            pltpu.VMEM((CB * C,), jnp.int32),
            pltpu.VMEM((CB, 2 * E), jnp.float32),
            pltpu.VMEM((CB * C, 2 * E), jnp.float32),
            pltpu.VMEM((CB * C,), jnp.float32),
            pltpu.SemaphoreType.DMA,
        ],
    )(_sc_kernel)
    return k(t_half, t_off, c_half, c_off, ttab2, ctab2)


def kernel(target, context, target_table, context_table):
    if target.ndim == 2:
        target = jnp.squeeze(target, axis=1)
    out = _run(target, context.reshape(-1), target_table, context_table)
    return out.reshape(B, C)


# speedup vs baseline: 1.0306x; 1.0306x over previous
import functools

import jax
import jax.numpy as jnp
from jax import lax
from jax.experimental import pallas as pl
from jax.experimental.pallas import tpu as pltpu
from jax.experimental.pallas import tpu_sc as plsc

B = 16384
C = 5
E = 64
NC = 2
NS = 16
NW = NC * NS
B_PER_W = B // NW
CB = 128
N_CHUNK = B_PER_W // CB


def _sc_kernel(target_hbm, ctxf_hbm, ttab_hbm, ctab_hbm, out_hbm,
               tidx_v, cidx_v, w_rows, c_rows, out_v, sem):
    wid = lax.axis_index("s") * NC + lax.axis_index("c")
    base = wid * B_PER_W

    def chunk_body(ch, _):
        tbase = base + ch * CB

        pltpu.sync_copy(target_hbm.at[pl.ds(tbase, CB)], tidx_v)
        pltpu.sync_copy(ctxf_hbm.at[pl.ds(tbase * C, CB * C)], cidx_v)

        cps = [pltpu.async_copy(ttab_hbm.at[tidx_v], w_rows, sem)]
        for j in range(C):
            cps.append(pltpu.async_copy(
                ctab_hbm.at[cidx_v.at[pl.ds(j * CB, CB)]],
                c_rows.at[pl.ds(j * CB, CB)], sem))
        for cp in cps:
            cp.wait()

        iota16 = lax.iota(jnp.int32, 16)

        def group_body(g, _):
            b_vec = iota16 + g * 16
            b5 = b_vec * C

            def e_body(e, accs):
                evec = jnp.zeros((16,), jnp.int32) + e
                wv = plsc.load_gather(w_rows, [b_vec, evec])
                return tuple(
                    accs[c] + wv * plsc.load_gather(c_rows, [b5 + c, evec])
                    for c in range(C))

            accs = lax.fori_loop(
                0, E, e_body,
                tuple(jnp.zeros((16,), jnp.float32) for _ in range(C)),
                unroll=4)
            for c in range(C):
                plsc.store_scatter(out_v, [b5 + c], accs[c])
            return 0

        lax.fori_loop(0, CB // 16, group_body, 0)
        pltpu.sync_copy(out_v, out_hbm.at[pl.ds(tbase * C, CB * C)])
        return 0

    lax.fori_loop(0, N_CHUNK, chunk_body, 0)


@jax.jit
def _run(target, ctx_flat, target_table, context_table):
    mesh = plsc.VectorSubcoreMesh(core_axis_name="c", subcore_axis_name="s")
    k = functools.partial(
        pl.kernel, mesh=mesh,
        out_type=jax.ShapeDtypeStruct((B * C,), jnp.float32),
        compiler_params=pltpu.CompilerParams(
            needs_layout_passes=False, use_tc_tiling_on_sc=False),
        scratch_types=[
            pltpu.VMEM((CB,), jnp.int32),
            pltpu.VMEM((CB * C,), jnp.int32),
            pltpu.VMEM((CB, E), jnp.float32),
            pltpu.VMEM((CB * C, E), jnp.float32),
            pltpu.VMEM((CB * C,), jnp.float32),
            pltpu.SemaphoreType.DMA,
        ],
    )(_sc_kernel)
    return k(target, ctx_flat, target_table, context_table)


def kernel(target, context, target_table, context_table):
    if target.ndim == 2:
        target = jnp.squeeze(target, axis=1)
    out = _run(target, context.reshape(-1), target_table, context_table)
    return out.reshape(B, C)


# speedup vs baseline: 1.0360x; 1.0052x over previous
import functools

import jax
import jax.numpy as jnp
from jax import lax
from jax.experimental import pallas as pl
from jax.experimental.pallas import tpu as pltpu
from jax.experimental.pallas import tpu_sc as plsc

B = 16384
C = 5
E = 64
NC = 2
NS = 16
NW = NC * NS
B_PER_W = B // NW
CB = 128
N_CHUNK = B_PER_W // CB


def _sc_kernel(target_hbm, ctxf_hbm, ttab_hbm, ctab_hbm, out_hbm,
               tidx_v, cidx_v, w_rows, c_rows, out_v, sem):
    wid = lax.axis_index("s") * NC + lax.axis_index("c")
    base = wid * B_PER_W

    def stage(ch, slot):
        tbase = base + ch * CB
        pltpu.sync_copy(target_hbm.at[pl.ds(tbase, CB)], tidx_v.at[slot])
        pltpu.sync_copy(ctxf_hbm.at[pl.ds(tbase * C, CB * C)],
                        cidx_v.at[slot])
        cps = [pltpu.async_copy(ttab_hbm.at[tidx_v.at[slot]],
                                w_rows.at[slot], sem.at[slot])]
        for j in range(C):
            cps.append(pltpu.async_copy(
                ctab_hbm.at[cidx_v.at[slot, pl.ds(j * CB, CB)]],
                c_rows.at[slot, pl.ds(j * CB, CB)], sem.at[slot]))
        return cps

    def compute(ch, slot, cps):
        tbase = base + ch * CB
        for cp in cps:
            cp.wait()
        iota16 = lax.iota(jnp.int32, 16)

        def group_body(g, _):
            b_vec = iota16 + g * 16
            b5 = b_vec * C

            def e_body(e, accs):
                evec = jnp.zeros((16,), jnp.int32) + e
                wv = plsc.load_gather(w_rows.at[slot], [b_vec, evec])
                return tuple(
                    accs[c] + wv * plsc.load_gather(
                        c_rows.at[slot], [b5 + c, evec])
                    for c in range(C))

            accs = lax.fori_loop(
                0, E, e_body,
                tuple(jnp.zeros((16,), jnp.float32) for _ in range(C)),
                unroll=4)
            for c in range(C):
                plsc.store_scatter(out_v.at[slot], [b5 + c], accs[c])
            return 0

        lax.fori_loop(0, CB // 16, group_body, 0)
        pltpu.sync_copy(out_v.at[slot],
                        out_hbm.at[pl.ds(tbase * C, CB * C)])

    cps = stage(0, 0)
    for ch in range(N_CHUNK):
        nxt = stage(ch + 1, (ch + 1) % 2) if ch + 1 < N_CHUNK else None
        compute(ch, ch % 2, cps)
        cps = nxt


@jax.jit
def _run(target, ctx_flat, target_table, context_table):
    mesh = plsc.VectorSubcoreMesh(core_axis_name="c", subcore_axis_name="s")
    k = functools.partial(
        pl.kernel, mesh=mesh,
        out_type=jax.ShapeDtypeStruct((B * C,), jnp.float32),
        compiler_params=pltpu.CompilerParams(
            needs_layout_passes=False, use_tc_tiling_on_sc=False),
        scratch_types=[
            pltpu.VMEM((2, CB), jnp.int32),
            pltpu.VMEM((2, CB * C), jnp.int32),
            pltpu.VMEM((2, CB, E), jnp.float32),
            pltpu.VMEM((2, CB * C, E), jnp.float32),
            pltpu.VMEM((2, CB * C), jnp.float32),
            pltpu.SemaphoreType.DMA((2,)),
        ],
    )(_sc_kernel)
    return k(target, ctx_flat, target_table, context_table)


def kernel(target, context, target_table, context_table):
    if target.ndim == 2:
        target = jnp.squeeze(target, axis=1)
    out = _run(target, context.reshape(-1), target_table, context_table)
    return out.reshape(B, C)
